# Initial kernel scaffold; baseline (speedup 1.0000x reference)
import jax
import jax.numpy as jnp
from jax.experimental import pallas as pl


def kernel(x, edge_index, batch, W1, b1, W2, b2):
    raise NotImplementedError("write your pallas kernel here")


# speedup vs baseline: 20.4020x; 20.4020x over previous
import functools

import jax
import jax.numpy as jnp
from jax import lax
from jax.experimental import pallas as pl
from jax.experimental.pallas import tpu as pltpu
from jax.experimental.pallas import tpu_sc as plsc

N = 100000
E = 3200000
G = 64
F = 16

NC = 2
NS = 16
NW = NC * NS

CH = 128
EPW = ((E + NW * CH - 1) // (NW * CH)) * CH
NE_PAD = EPW * NW
NCH = EPW // CH

RPT = 6256
N_PAD = RPT * NS

BN = 4000
GRID = N // BN


def _mesh():
    return plsc.VectorSubcoreMesh(core_axis_name="c", subcore_axis_name="s")


_SC_PARAMS = pltpu.CompilerParams(use_tc_tiling_on_sc=False)


def _sc_hist(colp, aux):
    @functools.partial(
        pl.kernel,
        out_type=jax.ShapeDtypeStruct((NC, N_PAD, F), jnp.float32),
        mesh=_mesh(),
        compiler_params=_SC_PARAMS,
        scratch_types=[
            pltpu.VMEM((1, CH), jnp.int32),
            pltpu.VMEM((CH, F), jnp.float32),
            pltpu.VMEM_SHARED((N_PAD, F), jnp.float32),
        ],
    )
    def k(col_hbm, aux_hbm, out_hbm, cbuf, gbuf, acc):
        c = lax.axis_index("c")
        s = lax.axis_index("s")
        pltpu.sync_copy(aux_hbm.at[pl.ds(0, RPT)], acc.at[pl.ds(s * RPT, RPT)])
        pltpu.sync_copy(aux_hbm.at[pl.ds(RPT, CH)], gbuf)
        plsc.subcore_barrier()
        base = (c * NS + s) * EPW

        @pl.loop(0, NCH)
        def _(j):
            e0 = base + j * CH
            pltpu.sync_copy(col_hbm.at[pl.ds(e0, CH)], cbuf.at[0])
            pltpu.sync_copy(gbuf, acc.at[cbuf.at[0]], add=True)

        plsc.subcore_barrier()
        pltpu.sync_copy(acc.at[pl.ds(s * RPT, RPT)],
                        out_hbm.at[c, pl.ds(s * RPT, RPT)])

    return k(colp, aux)


def _sc_agg(z, rowp, colp, aux):
    @functools.partial(
        pl.kernel,
        out_type=jax.ShapeDtypeStruct((NC, N_PAD, F), jnp.float32),
        mesh=_mesh(),
        compiler_params=_SC_PARAMS,
        scratch_types=[
            pltpu.VMEM((1, CH), jnp.int32),
            pltpu.VMEM((1, CH), jnp.int32),
            pltpu.VMEM((CH, F), jnp.float32),
            pltpu.VMEM_SHARED((N_PAD, F), jnp.float32),
        ],
    )
    def k(z_hbm, row_hbm, col_hbm, aux_hbm, out_hbm, cbuf, rbuf, gbuf, acc):
        c = lax.axis_index("c")
        s = lax.axis_index("s")
        pltpu.sync_copy(aux_hbm.at[pl.ds(0, RPT)], acc.at[pl.ds(s * RPT, RPT)])
        plsc.subcore_barrier()
        base = (c * NS + s) * EPW

        @pl.loop(0, NCH)
        def _(j):
            e0 = base + j * CH
            pltpu.sync_copy(col_hbm.at[pl.ds(e0, CH)], cbuf.at[0])
            pltpu.sync_copy(row_hbm.at[pl.ds(e0, CH)], rbuf.at[0])
            pltpu.sync_copy(z_hbm.at[rbuf.at[0]], gbuf)
            pltpu.sync_copy(gbuf, acc.at[cbuf.at[0]], add=True)

        plsc.subcore_barrier()
        pltpu.sync_copy(acc.at[pl.ds(s * RPT, RPT)],
                        out_hbm.at[c, pl.ds(s * RPT, RPT)])

    return k(z, rowp, colp, aux)


def _mm(a, w, kdim):
    out = a[:, 0:1] * w[0:1, :]
    for k in range(1, kdim):
        out = out + a[:, k:k + 1] * w[k:k + 1, :]
    return out


def _tc_pre(hist, x, W1):
    def body(h_ref, x_ref, w_ref, dis_ref, z_ref):
        d = h_ref[0, :, 0:1] + h_ref[1, :, 0:1] + 1.0
        dis = lax.rsqrt(d)
        dis_ref[...] = dis
        z_ref[...] = _mm(x_ref[...], w_ref[...], 5) * dis

    return pl.pallas_call(
        body,
        grid=(GRID,),
        in_specs=[
            pl.BlockSpec((NC, BN, F), lambda i: (0, i, 0)),
            pl.BlockSpec((BN, 5), lambda i: (i, 0)),
            pl.BlockSpec((5, F), lambda i: (0, 0)),
        ],
        out_specs=[
            pl.BlockSpec((BN, 1), lambda i: (i, 0)),
            pl.BlockSpec((BN, F), lambda i: (i, 0)),
        ],
        out_shape=[
            jax.ShapeDtypeStruct((N, 1), jnp.float32),
            jax.ShapeDtypeStruct((N, F), jnp.float32),
        ],
    )(hist, x, W1)


def _tc_mid(p, z, dis, W2, b1):
    def body(p_ref, z_ref, dis_ref, w_ref, b_ref, z2_ref):
        acc = p_ref[0] + p_ref[1] + z_ref[...]
        h = jnp.maximum(acc * dis_ref[...] + b_ref[...], 0.0)
        z2_ref[...] = _mm(h, w_ref[...], F) * dis_ref[...]

    return pl.pallas_call(
        body,
        grid=(GRID,),
        in_specs=[
            pl.BlockSpec((NC, BN, F), lambda i: (0, i, 0)),
            pl.BlockSpec((BN, F), lambda i: (i, 0)),
            pl.BlockSpec((BN, 1), lambda i: (i, 0)),
            pl.BlockSpec((F, F), lambda i: (0, 0)),
            pl.BlockSpec((1, F), lambda i: (0, 0)),
        ],
        out_specs=pl.BlockSpec((BN, F), lambda i: (i, 0)),
        out_shape=jax.ShapeDtypeStruct((N, F), jnp.float32),
    )(p, z, dis, W2, b1)


def _tc_post(p, z, dis, b2, bat):
    def body(p_ref, z_ref, dis_ref, b_ref, bat_ref, out_ref, s_acc, c_acc):
        i = pl.program_id(0)
        acc = p_ref[0] + p_ref[1] + z_ref[...]
        h = jnp.maximum(acc * dis_ref[...] + b_ref[...], 0.0)
        iota = lax.broadcasted_iota(jnp.int32, (G, BN), 0)
        eqf = (bat_ref[0] == iota).astype(jnp.float32)
        s_part = jax.lax.dot(eqf, h, precision=jax.lax.Precision.HIGHEST)
        c_part = jnp.broadcast_to(jnp.sum(eqf, axis=1, keepdims=True), (G, F))

        @pl.when(i == 0)
        def _():
            s_acc[...] = jnp.zeros_like(s_acc)
            c_acc[...] = jnp.zeros_like(c_acc)

        s_acc[...] += s_part
        c_acc[...] += c_part

        @pl.when(i == GRID - 1)
        def _():
            out_ref[...] = s_acc[...] / jnp.maximum(c_acc[...], 1.0)

    return pl.pallas_call(
        body,
        grid=(GRID,),
        in_specs=[
            pl.BlockSpec((NC, BN, F), lambda i: (0, i, 0)),
            pl.BlockSpec((BN, F), lambda i: (i, 0)),
            pl.BlockSpec((BN, 1), lambda i: (i, 0)),
            pl.BlockSpec((1, F), lambda i: (0, 0)),
            pl.BlockSpec((1, 1, BN), lambda i: (i, 0, 0)),
        ],
        out_specs=pl.BlockSpec((G, F), lambda i: (0, 0)),
        out_shape=jax.ShapeDtypeStruct((G, F), jnp.float32),
        scratch_shapes=[
            pltpu.VMEM((G, F), jnp.float32),
            pltpu.VMEM((G, F), jnp.float32),
        ],
    )(p, z, dis, b2, bat)


def kernel(x, edge_index, batch, W1, b1, W2, b2):
    row = edge_index[0]
    col = edge_index[1]
    pad = NE_PAD - E
    rowp = jnp.concatenate([row, jnp.zeros((pad,), jnp.int32)])
    colp = jnp.concatenate([col, jnp.full((pad,), N, jnp.int32)])
    aux = jnp.concatenate([jnp.zeros((RPT, F), jnp.float32),
                           jnp.ones((CH, F), jnp.float32)], axis=0)

    hist = _sc_hist(colp, aux)
    dis, z1 = _tc_pre(hist, x, W1)
    p1 = _sc_agg(z1, rowp, colp, aux)
    z2 = _tc_mid(p1, z1, dis, W2, b1.reshape(1, F))
    p2 = _sc_agg(z2, rowp, colp, aux)
    return _tc_post(p2, z2, dis, b2.reshape(1, F), batch.reshape(GRID, 1, BN))


# speedup vs baseline: 55.8707x; 2.7385x over previous
import functools

import jax
import jax.numpy as jnp
from jax import lax
from jax.experimental import pallas as pl
from jax.experimental.pallas import tpu as pltpu
from jax.experimental.pallas import tpu_sc as plsc

N = 100000
E = 3200000
G = 64
F = 16

NC = 2
NS = 16
NW = NC * NS

CH = 128
K = 8
EPW = ((E + NW * CH * K - 1) // (NW * CH * K)) * CH * K
NE_PAD = EPW * NW
NCH = EPW // CH
NG = NCH // K

RPT = 6256
N_PAD = RPT * NS

BN = 4000
GRID = N // BN


def _mesh():
    return plsc.VectorSubcoreMesh(core_axis_name="c", subcore_axis_name="s")


_SC_PARAMS = pltpu.CompilerParams(use_tc_tiling_on_sc=False)


def _sc_hist(colp, aux):
    @functools.partial(
        pl.kernel,
        out_type=jax.ShapeDtypeStruct((NC, N_PAD, F), jnp.float32),
        mesh=_mesh(),
        compiler_params=_SC_PARAMS,
        scratch_types=[
            pltpu.VMEM((K, CH), jnp.int32),
            pltpu.VMEM((CH, F), jnp.float32),
            pltpu.VMEM_SHARED((N_PAD, F), jnp.float32),
            pltpu.SemaphoreType.DMA,
            pltpu.SemaphoreType.DMA,
        ],
    )
    def k(col_hbm, aux_hbm, out_hbm, cbuf, gbuf, acc, semi, sems):
        c = lax.axis_index("c")
        s = lax.axis_index("s")
        pltpu.sync_copy(aux_hbm.at[pl.ds(0, RPT)], acc.at[pl.ds(s * RPT, RPT)])
        pltpu.sync_copy(aux_hbm.at[pl.ds(RPT, CH)], gbuf)
        plsc.subcore_barrier()
        base = (c * NS + s) * EPW

        @pl.loop(0, NG)
        def _(g):
            e0 = base + g * (K * CH)
            hc = [pltpu.async_copy(col_hbm.at[pl.ds(e0 + b * CH, CH)],
                                   cbuf.at[b], semi) for b in range(K)]
            hs = []
            for b in range(K):
                hc[b].wait()
                hs.append(pltpu.async_copy(gbuf, acc.at[cbuf.at[b]], sems,
                                           add=True))
            for h in hs:
                h.wait()

        plsc.subcore_barrier()
        pltpu.sync_copy(acc.at[pl.ds(s * RPT, RPT)],
                        out_hbm.at[c, pl.ds(s * RPT, RPT)])

    return k(colp, aux)


def _sc_agg(z, rowp, colp, aux):
    @functools.partial(
        pl.kernel,
        out_type=jax.ShapeDtypeStruct((NC, N_PAD, F), jnp.float32),
        mesh=_mesh(),
        compiler_params=_SC_PARAMS,
        scratch_types=[
            pltpu.VMEM((K, CH), jnp.int32),
            pltpu.VMEM((K, CH), jnp.int32),
            pltpu.VMEM((K, CH, F), jnp.float32),
            pltpu.VMEM_SHARED((N_PAD, F), jnp.float32),
            pltpu.SemaphoreType.DMA,
            pltpu.SemaphoreType.DMA,
            pltpu.SemaphoreType.DMA,
        ],
    )
    def k(z_hbm, row_hbm, col_hbm, aux_hbm, out_hbm,
          cbuf, rbuf, gbuf, acc, semi, semg, sems):
        c = lax.axis_index("c")
        s = lax.axis_index("s")
        pltpu.sync_copy(aux_hbm.at[pl.ds(0, RPT)], acc.at[pl.ds(s * RPT, RPT)])
        plsc.subcore_barrier()
        base = (c * NS + s) * EPW

        @pl.loop(0, NG)
        def _(g):
            e0 = base + g * (K * CH)
            hc = []
            hr = []
            for b in range(K):
                hc.append(pltpu.async_copy(col_hbm.at[pl.ds(e0 + b * CH, CH)],
                                           cbuf.at[b], semi))
                hr.append(pltpu.async_copy(row_hbm.at[pl.ds(e0 + b * CH, CH)],
                                           rbuf.at[b], semi))
            hg = []
            for b in range(K):
                hc[b].wait()
                hr[b].wait()
                hg.append(pltpu.async_copy(z_hbm.at[rbuf.at[b]],
                                           gbuf.at[b], semg))
            hs = []
            for b in range(K):
                hg[b].wait()
                hs.append(pltpu.async_copy(gbuf.at[b], acc.at[cbuf.at[b]],
                                           sems, add=True))
            for h in hs:
                h.wait()

        plsc.subcore_barrier()
        pltpu.sync_copy(acc.at[pl.ds(s * RPT, RPT)],
                        out_hbm.at[c, pl.ds(s * RPT, RPT)])

    return k(z, rowp, colp, aux)


def _mm(a, w, kdim):
    out = a[:, 0:1] * w[0:1, :]
    for k in range(1, kdim):
        out = out + a[:, k:k + 1] * w[k:k + 1, :]
    return out


def _tc_pre(hist, x, W1):
    def body(h_ref, x_ref, w_ref, dis_ref, z_ref):
        d = h_ref[0, :, 0:1] + h_ref[1, :, 0:1] + 1.0
        dis = lax.rsqrt(d)
        dis_ref[...] = dis
        z_ref[...] = _mm(x_ref[...], w_ref[...], 5) * dis

    return pl.pallas_call(
        body,
        grid=(GRID,),
        in_specs=[
            pl.BlockSpec((NC, BN, F), lambda i: (0, i, 0)),
            pl.BlockSpec((BN, 5), lambda i: (i, 0)),
            pl.BlockSpec((5, F), lambda i: (0, 0)),
        ],
        out_specs=[
            pl.BlockSpec((BN, 1), lambda i: (i, 0)),
            pl.BlockSpec((BN, F), lambda i: (i, 0)),
        ],
        out_shape=[
            jax.ShapeDtypeStruct((N, 1), jnp.float32),
            jax.ShapeDtypeStruct((N, F), jnp.float32),
        ],
    )(hist, x, W1)


def _tc_mid(p, z, dis, W2, b1):
    def body(p_ref, z_ref, dis_ref, w_ref, b_ref, z2_ref):
        acc = p_ref[0] + p_ref[1] + z_ref[...]
        h = jnp.maximum(acc * dis_ref[...] + b_ref[...], 0.0)
        z2_ref[...] = _mm(h, w_ref[...], F) * dis_ref[...]

    return pl.pallas_call(
        body,
        grid=(GRID,),
        in_specs=[
            pl.BlockSpec((NC, BN, F), lambda i: (0, i, 0)),
            pl.BlockSpec((BN, F), lambda i: (i, 0)),
            pl.BlockSpec((BN, 1), lambda i: (i, 0)),
            pl.BlockSpec((F, F), lambda i: (0, 0)),
            pl.BlockSpec((1, F), lambda i: (0, 0)),
        ],
        out_specs=pl.BlockSpec((BN, F), lambda i: (i, 0)),
        out_shape=jax.ShapeDtypeStruct((N, F), jnp.float32),
    )(p, z, dis, W2, b1)


def _tc_post(p, z, dis, b2, bat):
    def body(p_ref, z_ref, dis_ref, b_ref, bat_ref, out_ref, s_acc, c_acc):
        i = pl.program_id(0)
        acc = p_ref[0] + p_ref[1] + z_ref[...]
        h = jnp.maximum(acc * dis_ref[...] + b_ref[...], 0.0)
        iota = lax.broadcasted_iota(jnp.int32, (G, BN), 0)
        eqf = (bat_ref[0] == iota).astype(jnp.float32)
        s_part = jax.lax.dot(eqf, h, precision=jax.lax.Precision.HIGHEST)
        c_part = jnp.broadcast_to(jnp.sum(eqf, axis=1, keepdims=True), (G, F))

        @pl.when(i == 0)
        def _():
            s_acc[...] = jnp.zeros_like(s_acc)
            c_acc[...] = jnp.zeros_like(c_acc)

        s_acc[...] += s_part
        c_acc[...] += c_part

        @pl.when(i == GRID - 1)
        def _():
            out_ref[...] = s_acc[...] / jnp.maximum(c_acc[...], 1.0)

    return pl.pallas_call(
        body,
        grid=(GRID,),
        in_specs=[
            pl.BlockSpec((NC, BN, F), lambda i: (0, i, 0)),
            pl.BlockSpec((BN, F), lambda i: (i, 0)),
            pl.BlockSpec((BN, 1), lambda i: (i, 0)),
            pl.BlockSpec((1, F), lambda i: (0, 0)),
            pl.BlockSpec((1, 1, BN), lambda i: (i, 0, 0)),
        ],
        out_specs=pl.BlockSpec((G, F), lambda i: (0, 0)),
        out_shape=jax.ShapeDtypeStruct((G, F), jnp.float32),
        scratch_shapes=[
            pltpu.VMEM((G, F), jnp.float32),
            pltpu.VMEM((G, F), jnp.float32),
        ],
    )(p, z, dis, b2, bat)


def kernel(x, edge_index, batch, W1, b1, W2, b2):
    row = edge_index[0]
    col = edge_index[1]
    pad = NE_PAD - E
    rowp = jnp.concatenate([row, jnp.zeros((pad,), jnp.int32)])
    colp = jnp.concatenate([col, jnp.full((pad,), N, jnp.int32)])
    aux = jnp.concatenate([jnp.zeros((RPT, F), jnp.float32),
                           jnp.ones((CH, F), jnp.float32)], axis=0)

    hist = _sc_hist(colp, aux)
    dis, z1 = _tc_pre(hist, x, W1)
    p1 = _sc_agg(z1, rowp, colp, aux)
    z2 = _tc_mid(p1, z1, dis, W2, b1.reshape(1, F))
    p2 = _sc_agg(z2, rowp, colp, aux)
    return _tc_post(p2, z2, dis, b2.reshape(1, F), batch.reshape(GRID, 1, BN))


# speedup vs baseline: 101.7936x; 1.8219x over previous
import functools

import jax
import jax.numpy as jnp
from jax import lax
from jax.experimental import pallas as pl
from jax.experimental.pallas import tpu as pltpu
from jax.experimental.pallas import tpu_sc as plsc

N = 100000
E = 3200000
G = 64
F = 16

NC = 2
NS = 16
NW = NC * NS

CH = 128
K = 8
NCHT = E // CH
NGT = NCHT // K

RPT = 6256
N_PAD = RPT * NS
R_PAD = N_PAD // 8

BN8 = 736
GRID = R_PAD // BN8

NPCH = N_PAD // CH
GA = 80

_HI = jax.lax.Precision.HIGHEST


def _mesh():
    return plsc.VectorSubcoreMesh(core_axis_name="c", subcore_axis_name="s")


_SC_PARAMS = pltpu.CompilerParams(use_tc_tiling_on_sc=False)


def _sc_hist(col, aux):
    @functools.partial(
        pl.kernel,
        out_type=jax.ShapeDtypeStruct((NC, N_PAD, F), jnp.float32),
        mesh=_mesh(),
        compiler_params=_SC_PARAMS,
        scratch_types=[
            pltpu.VMEM((K, CH), jnp.int32),
            pltpu.VMEM((CH, F), jnp.float32),
            pltpu.VMEM_SHARED((N_PAD, F), jnp.float32),
            pltpu.SemaphoreType.DMA,
            pltpu.SemaphoreType.DMA,
        ],
    )
    def k(col_hbm, aux_hbm, out_hbm, cbuf, gbuf, acc, semi, sems):
        c = lax.axis_index("c")
        s = lax.axis_index("s")
        wid = c * NS + s
        pltpu.sync_copy(aux_hbm.at[pl.ds(0, RPT)], acc.at[pl.ds(s * RPT, RPT)])
        pltpu.sync_copy(aux_hbm.at[pl.ds(RPT, CH)], gbuf)
        plsc.subcore_barrier()
        glo = wid * NGT // NW
        ghi = (wid + 1) * NGT // NW

        @pl.loop(glo, ghi)
        def _(g):
            e0 = g * (K * CH)
            hc = [pltpu.async_copy(col_hbm.at[pl.ds(e0 + b * CH, CH)],
                                   cbuf.at[b], semi) for b in range(K)]
            hs = []
            for b in range(K):
                hc[b].wait()
                hs.append(pltpu.async_copy(gbuf, acc.at[cbuf.at[b]], sems,
                                           add=True))
            for h in hs:
                h.wait()

        plsc.subcore_barrier()
        pltpu.sync_copy(acc.at[pl.ds(s * RPT, RPT)],
                        out_hbm.at[c, pl.ds(s * RPT, RPT)])

    return k(col, aux)


def _sc_agg(zfeat, row, col, aux):
    @functools.partial(
        pl.kernel,
        out_type=jax.ShapeDtypeStruct((NC, N_PAD, F), jnp.float32),
        mesh=_mesh(),
        compiler_params=_SC_PARAMS,
        scratch_types=[
            pltpu.VMEM((K, CH), jnp.int32),
            pltpu.VMEM((K, CH), jnp.int32),
            pltpu.VMEM((K, CH, F), jnp.float32),
            pltpu.VMEM_SHARED((N_PAD, F), jnp.float32),
            pltpu.SemaphoreType.DMA,
            pltpu.SemaphoreType.DMA,
            pltpu.SemaphoreType.DMA,
        ],
    )
    def k(z_hbm, row_hbm, col_hbm, aux_hbm, out_hbm,
          cbuf, rbuf, gbuf, acc, semi, semg, sems):
        c = lax.axis_index("c")
        s = lax.axis_index("s")
        wid = c * NS + s
        pltpu.sync_copy(aux_hbm.at[pl.ds(0, RPT)], acc.at[pl.ds(s * RPT, RPT)])
        plsc.subcore_barrier()
        glo = wid * NGT // NW
        ghi = (wid + 1) * NGT // NW

        @pl.loop(glo, ghi)
        def _(g):
            e0 = g * (K * CH)
            hc = []
            hr = []
            for b in range(K):
                hc.append(pltpu.async_copy(col_hbm.at[pl.ds(e0 + b * CH, CH)],
                                           cbuf.at[b], semi))
                hr.append(pltpu.async_copy(row_hbm.at[pl.ds(e0 + b * CH, CH)],
                                           rbuf.at[b], semi))
            hg = []
            for b in range(K):
                hc[b].wait()
                hr[b].wait()
                hg.append(pltpu.async_copy(z_hbm.at[rbuf.at[b]],
                                           gbuf.at[b], semg))
            hs = []
            for b in range(K):
                hg[b].wait()
                hs.append(pltpu.async_copy(gbuf.at[b], acc.at[cbuf.at[b]],
                                           sems, add=True))
            for h in hs:
                h.wait()

        plsc.subcore_barrier()
        pltpu.sync_copy(acc.at[pl.ds(s * RPT, RPT)],
                        out_hbm.at[c, pl.ds(s * RPT, RPT)])

    return k(zfeat, row, col, aux)


def _sc_pool(h2feat, batp, aux):
    @functools.partial(
        pl.kernel,
        out_type=jax.ShapeDtypeStruct((NC, 2, GA, F), jnp.float32),
        mesh=_mesh(),
        compiler_params=_SC_PARAMS,
        scratch_types=[
            pltpu.VMEM((K, CH), jnp.int32),
            pltpu.VMEM((K, CH, F), jnp.float32),
            pltpu.VMEM((CH, F), jnp.float32),
            pltpu.VMEM_SHARED((GA, F), jnp.float32),
            pltpu.VMEM_SHARED((GA, F), jnp.float32),
            pltpu.SemaphoreType.DMA,
            pltpu.SemaphoreType.DMA,
            pltpu.SemaphoreType.DMA,
        ],
    )
    def k(h_hbm, bat_hbm, aux_hbm, out_hbm,
          cbuf, gbuf, obuf, accs, accc, semi, semg, sems):
        c = lax.axis_index("c")
        s = lax.axis_index("s")
        wid = c * NS + s
        pltpu.sync_copy(aux_hbm.at[pl.ds(RPT, CH)], obuf)

        @pl.when(s == 0)
        def _():
            pltpu.sync_copy(aux_hbm.at[pl.ds(0, GA)], accs)
            pltpu.sync_copy(aux_hbm.at[pl.ds(0, GA)], accc)

        plsc.subcore_barrier()
        jlo = wid * NPCH // NW
        jhi = (wid + 1) * NPCH // NW

        @pl.loop(jlo, jhi)
        def _(j):
            hc = pltpu.async_copy(bat_hbm.at[pl.ds(j * CH, CH)],
                                  cbuf.at[0], semi)
            hh = pltpu.async_copy(h_hbm.at[pl.ds(j * CH, CH)],
                                  gbuf.at[0], semg)
            hc.wait()
            hh.wait()
            h1 = pltpu.async_copy(gbuf.at[0], accs.at[cbuf.at[0]],
                                  sems, add=True)
            h2 = pltpu.async_copy(obuf, accc.at[cbuf.at[0]],
                                  sems, add=True)
            h1.wait()
            h2.wait()

        plsc.subcore_barrier()

        @pl.when(s == 0)
        def _():
            pltpu.sync_copy(accs, out_hbm.at[c, 0])
            pltpu.sync_copy(accc, out_hbm.at[c, 1])

    return k(h2feat, batp, aux)


def _tc_pre(histl, xlin, w1k):
    def body(h_ref, x_ref, w_ref, dis_ref, z_ref):
        dis = lax.rsqrt(h_ref[0] + h_ref[1] + 1.0)
        dis_ref[...] = dis
        z_ref[...] = jax.lax.dot(x_ref[...], w_ref[...], precision=_HI) * dis

    return pl.pallas_call(
        body,
        grid=(GRID,),
        in_specs=[
            pl.BlockSpec((NC, BN8, 128), lambda i: (0, i, 0)),
            pl.BlockSpec((BN8, 40), lambda i: (i, 0)),
            pl.BlockSpec((40, 128), lambda i: (0, 0)),
        ],
        out_specs=[
            pl.BlockSpec((BN8, 128), lambda i: (i, 0)),
            pl.BlockSpec((BN8, 128), lambda i: (i, 0)),
        ],
        out_shape=[
            jax.ShapeDtypeStruct((R_PAD, 128), jnp.float32),
            jax.ShapeDtypeStruct((R_PAD, 128), jnp.float32),
        ],
    )(histl, xlin, w1k)


def _tc_mid(p, z, dis, w2k, b1l):
    def body(p_ref, z_ref, dis_ref, w_ref, b_ref, z2_ref):
        acc = p_ref[0] + p_ref[1] + z_ref[...]
        h = jnp.maximum(acc * dis_ref[...] + b_ref[...], 0.0)
        z2_ref[...] = jax.lax.dot(h, w_ref[...], precision=_HI) * dis_ref[...]

    return pl.pallas_call(
        body,
        grid=(GRID,),
        in_specs=[
            pl.BlockSpec((NC, BN8, 128), lambda i: (0, i, 0)),
            pl.BlockSpec((BN8, 128), lambda i: (i, 0)),
            pl.BlockSpec((BN8, 128), lambda i: (i, 0)),
            pl.BlockSpec((128, 128), lambda i: (0, 0)),
            pl.BlockSpec((1, 128), lambda i: (0, 0)),
        ],
        out_specs=pl.BlockSpec((BN8, 128), lambda i: (i, 0)),
        out_shape=jax.ShapeDtypeStruct((R_PAD, 128), jnp.float32),
    )(p, z, dis, w2k, b1l)


def _tc_fin(p, z, dis, b2l):
    def body(p_ref, z_ref, dis_ref, b_ref, h_ref):
        acc = p_ref[0] + p_ref[1] + z_ref[...]
        h_ref[...] = jnp.maximum(acc * dis_ref[...] + b_ref[...], 0.0)

    return pl.pallas_call(
        body,
        grid=(GRID,),
        in_specs=[
            pl.BlockSpec((NC, BN8, 128), lambda i: (0, i, 0)),
            pl.BlockSpec((BN8, 128), lambda i: (i, 0)),
            pl.BlockSpec((BN8, 128), lambda i: (i, 0)),
            pl.BlockSpec((1, 128), lambda i: (0, 0)),
        ],
        out_specs=pl.BlockSpec((BN8, 128), lambda i: (i, 0)),
        out_shape=jax.ShapeDtypeStruct((R_PAD, 128), jnp.float32),
    )(p, z, dis, b2l)


def _tc_div(pp):
    def body(p_ref, out_ref):
        s = p_ref[0, 0] + p_ref[1, 0]
        cnt = p_ref[0, 1] + p_ref[1, 1]
        out_ref[...] = (s / jnp.maximum(cnt, 1.0))[:G, :]

    return pl.pallas_call(
        body,
        grid=(1,),
        in_specs=[pl.BlockSpec((NC, 2, GA, F), lambda i: (0, 0, 0, 0))],
        out_specs=pl.BlockSpec((G, F), lambda i: (0, 0)),
        out_shape=jax.ShapeDtypeStruct((G, F), jnp.float32),
    )(pp)


def kernel(x, edge_index, batch, W1, b1, W2, b2):
    row = edge_index[0]
    col = edge_index[1]
    batp = jnp.concatenate([batch, jnp.full((N_PAD - N,), G, jnp.int32)])
    aux = jnp.concatenate([jnp.zeros((RPT, F), jnp.float32),
                           jnp.ones((CH, F), jnp.float32)], axis=0)
    eye8 = jnp.eye(8, dtype=jnp.float32)
    w1k = jnp.kron(eye8, W1)
    w2k = jnp.kron(eye8, W2)
    b1l = jnp.tile(b1, 8).reshape(1, 128)
    b2l = jnp.tile(b2, 8).reshape(1, 128)
    xlin = jnp.concatenate(
        [x, jnp.zeros((N_PAD - N, 5), jnp.float32)]).reshape(R_PAD, 40)

    hist = _sc_hist(col, aux)
    histl = hist.reshape(NC, R_PAD, 128)
    dis, z1 = _tc_pre(histl, xlin, w1k)
    p1 = _sc_agg(z1.reshape(N_PAD, F), row, col, aux)
    z2 = _tc_mid(p1.reshape(NC, R_PAD, 128), z1, dis, w2k, b1l)
    p2 = _sc_agg(z2.reshape(N_PAD, F), row, col, aux)
    h2 = _tc_fin(p2.reshape(NC, R_PAD, 128), z2, dis, b2l)
    pp = _sc_pool(h2.reshape(N_PAD, F), batp, aux)
    return _tc_div(pp)


# speedup vs baseline: 110.4371x; 1.0849x over previous
import functools

import jax
import jax.numpy as jnp
from jax import lax
from jax.experimental import pallas as pl
from jax.experimental.pallas import tpu as pltpu
from jax.experimental.pallas import tpu_sc as plsc

N = 100000
E = 3200000
G = 64
F = 16

NC = 2
NS = 16
NW = NC * NS

CH = 128
K = 10
KH = 20
NCHT = E // CH
NGT = NCHT // K
NGTH = NCHT // KH

RPT = 6256
N_PAD = RPT * NS
R_PAD = N_PAD // 8

BN8 = 736
GRID = R_PAD // BN8

NPCH = N_PAD // CH
GA = 80

_HI = jax.lax.Precision.HIGHEST


def _mesh():
    return plsc.VectorSubcoreMesh(core_axis_name="c", subcore_axis_name="s")


_SC_PARAMS = pltpu.CompilerParams(use_tc_tiling_on_sc=False)


def _sc_hist(ei, aux):
    @functools.partial(
        pl.kernel,
        out_type=jax.ShapeDtypeStruct((NC, N_PAD, F), jnp.float32),
        mesh=_mesh(),
        compiler_params=_SC_PARAMS,
        scratch_types=[
            pltpu.VMEM((KH, CH), jnp.int32),
            pltpu.VMEM((CH, F), jnp.float32),
            pltpu.VMEM_SHARED((N_PAD, F), jnp.float32),
            pltpu.SemaphoreType.DMA,
            pltpu.SemaphoreType.DMA,
        ],
    )
    def k(ei_hbm, aux_hbm, out_hbm, cbuf, gbuf, acc, semi, sems):
        c = lax.axis_index("c")
        s = lax.axis_index("s")
        wid = c * NS + s
        pltpu.sync_copy(aux_hbm.at[pl.ds(0, RPT)], acc.at[pl.ds(s * RPT, RPT)])
        pltpu.sync_copy(aux_hbm.at[pl.ds(RPT, CH)], gbuf)
        plsc.subcore_barrier()
        glo = wid * NGTH // NW
        ghi = (wid + 1) * NGTH // NW

        @pl.loop(glo, ghi)
        def _(g):
            pltpu.async_copy(ei_hbm.at[1, pl.ds(g * KH, KH), :],
                             cbuf, semi).wait()
            hs = [pltpu.async_copy(gbuf, acc.at[cbuf.at[b]], sems, add=True)
                  for b in range(KH)]
            for h in hs:
                h.wait()

        plsc.subcore_barrier()
        pltpu.sync_copy(acc.at[pl.ds(s * RPT, RPT)],
                        out_hbm.at[c, pl.ds(s * RPT, RPT)])

    return k(ei, aux)


def _sc_agg(zfeat, ei, aux):
    @functools.partial(
        pl.kernel,
        out_type=jax.ShapeDtypeStruct((NC, N_PAD, F), jnp.float32),
        mesh=_mesh(),
        compiler_params=_SC_PARAMS,
        scratch_types=[
            pltpu.VMEM((K, CH), jnp.int32),
            pltpu.VMEM((K, CH), jnp.int32),
            pltpu.VMEM((K, CH, F), jnp.float32),
            pltpu.VMEM_SHARED((N_PAD, F), jnp.float32),
            pltpu.SemaphoreType.DMA,
            pltpu.SemaphoreType.DMA,
            pltpu.SemaphoreType.DMA,
        ],
    )
    def k(z_hbm, ei_hbm, aux_hbm, out_hbm,
          cbuf, rbuf, gbuf, acc, semi, semg, sems):
        c = lax.axis_index("c")
        s = lax.axis_index("s")
        wid = c * NS + s
        pltpu.sync_copy(aux_hbm.at[pl.ds(0, RPT)], acc.at[pl.ds(s * RPT, RPT)])
        plsc.subcore_barrier()
        glo = wid * NGT // NW
        ghi = (wid + 1) * NGT // NW

        @pl.loop(glo, ghi)
        def _(g):
            hc = pltpu.async_copy(ei_hbm.at[1, pl.ds(g * K, K), :],
                                  cbuf, semi)
            hr = pltpu.async_copy(ei_hbm.at[0, pl.ds(g * K, K), :],
                                  rbuf, semi)
            hr.wait()
            hg = [pltpu.async_copy(z_hbm.at[rbuf.at[b]], gbuf.at[b], semg)
                  for b in range(K)]
            hc.wait()
            hs = []
            for b in range(K):
                hg[b].wait()
                hs.append(pltpu.async_copy(gbuf.at[b], acc.at[cbuf.at[b]],
                                           sems, add=True))
            for h in hs:
                h.wait()

        plsc.subcore_barrier()
        pltpu.sync_copy(acc.at[pl.ds(s * RPT, RPT)],
                        out_hbm.at[c, pl.ds(s * RPT, RPT)])

    return k(zfeat, ei, aux)


def _sc_pool(h2feat, batp, aux):
    @functools.partial(
        pl.kernel,
        out_type=jax.ShapeDtypeStruct((NC, 2, GA, F), jnp.float32),
        mesh=_mesh(),
        compiler_params=_SC_PARAMS,
        scratch_types=[
            pltpu.VMEM((K, CH), jnp.int32),
            pltpu.VMEM((K, CH, F), jnp.float32),
            pltpu.VMEM((CH, F), jnp.float32),
            pltpu.VMEM_SHARED((GA, F), jnp.float32),
            pltpu.VMEM_SHARED((GA, F), jnp.float32),
            pltpu.SemaphoreType.DMA,
            pltpu.SemaphoreType.DMA,
            pltpu.SemaphoreType.DMA,
        ],
    )
    def k(h_hbm, bat_hbm, aux_hbm, out_hbm,
          cbuf, gbuf, obuf, accs, accc, semi, semg, sems):
        c = lax.axis_index("c")
        s = lax.axis_index("s")
        wid = c * NS + s
        pltpu.sync_copy(aux_hbm.at[pl.ds(RPT, CH)], obuf)

        @pl.when(s == 0)
        def _():
            pltpu.sync_copy(aux_hbm.at[pl.ds(0, GA)], accs)
            pltpu.sync_copy(aux_hbm.at[pl.ds(0, GA)], accc)

        plsc.subcore_barrier()
        jlo = wid * NPCH // NW
        jhi = (wid + 1) * NPCH // NW

        @pl.loop(jlo, jhi)
        def _(j):
            hc = pltpu.async_copy(bat_hbm.at[pl.ds(j * CH, CH)],
                                  cbuf.at[0], semi)
            hh = pltpu.async_copy(h_hbm.at[pl.ds(j * CH, CH)],
                                  gbuf.at[0], semg)
            hc.wait()
            hh.wait()
            h1 = pltpu.async_copy(gbuf.at[0], accs.at[cbuf.at[0]],
                                  sems, add=True)
            h2 = pltpu.async_copy(obuf, accc.at[cbuf.at[0]],
                                  sems, add=True)
            h1.wait()
            h2.wait()

        plsc.subcore_barrier()

        @pl.when(s == 0)
        def _():
            pltpu.sync_copy(accs, out_hbm.at[c, 0])
            pltpu.sync_copy(accc, out_hbm.at[c, 1])

    return k(h2feat, batp, aux)


def _tc_pre(histl, xlin, w1k):
    def body(h_ref, x_ref, w_ref, dis_ref, z_ref):
        dis = lax.rsqrt(h_ref[0] + h_ref[1] + 1.0)
        dis_ref[...] = dis
        z_ref[...] = jax.lax.dot(x_ref[...], w_ref[...], precision=_HI) * dis

    return pl.pallas_call(
        body,
        grid=(GRID,),
        in_specs=[
            pl.BlockSpec((NC, BN8, 128), lambda i: (0, i, 0)),
            pl.BlockSpec((BN8, 40), lambda i: (i, 0)),
            pl.BlockSpec((40, 128), lambda i: (0, 0)),
        ],
        out_specs=[
            pl.BlockSpec((BN8, 128), lambda i: (i, 0)),
            pl.BlockSpec((BN8, 128), lambda i: (i, 0)),
        ],
        out_shape=[
            jax.ShapeDtypeStruct((R_PAD, 128), jnp.float32),
            jax.ShapeDtypeStruct((R_PAD, 128), jnp.float32),
        ],
    )(histl, xlin, w1k)


def _tc_mid(p, z, dis, w2k, b1l):
    def body(p_ref, z_ref, dis_ref, w_ref, b_ref, z2_ref):
        acc = p_ref[0] + p_ref[1] + z_ref[...]
        h = jnp.maximum(acc * dis_ref[...] + b_ref[...], 0.0)
        z2_ref[...] = jax.lax.dot(h, w_ref[...], precision=_HI) * dis_ref[...]

    return pl.pallas_call(
        body,
        grid=(GRID,),
        in_specs=[
            pl.BlockSpec((NC, BN8, 128), lambda i: (0, i, 0)),
            pl.BlockSpec((BN8, 128), lambda i: (i, 0)),
            pl.BlockSpec((BN8, 128), lambda i: (i, 0)),
            pl.BlockSpec((128, 128), lambda i: (0, 0)),
            pl.BlockSpec((1, 128), lambda i: (0, 0)),
        ],
        out_specs=pl.BlockSpec((BN8, 128), lambda i: (i, 0)),
        out_shape=jax.ShapeDtypeStruct((R_PAD, 128), jnp.float32),
    )(p, z, dis, w2k, b1l)


def _tc_fin(p, z, dis, b2l):
    def body(p_ref, z_ref, dis_ref, b_ref, h_ref):
        acc = p_ref[0] + p_ref[1] + z_ref[...]
        h_ref[...] = jnp.maximum(acc * dis_ref[...] + b_ref[...], 0.0)

    return pl.pallas_call(
        body,
        grid=(GRID,),
        in_specs=[
            pl.BlockSpec((NC, BN8, 128), lambda i: (0, i, 0)),
            pl.BlockSpec((BN8, 128), lambda i: (i, 0)),
            pl.BlockSpec((BN8, 128), lambda i: (i, 0)),
            pl.BlockSpec((1, 128), lambda i: (0, 0)),
        ],
        out_specs=pl.BlockSpec((BN8, 128), lambda i: (i, 0)),
        out_shape=jax.ShapeDtypeStruct((R_PAD, 128), jnp.float32),
    )(p, z, dis, b2l)


def _tc_div(pp):
    def body(p_ref, out_ref):
        s = p_ref[0, 0] + p_ref[1, 0]
        cnt = p_ref[0, 1] + p_ref[1, 1]
        out_ref[...] = (s / jnp.maximum(cnt, 1.0))[:G, :]

    return pl.pallas_call(
        body,
        grid=(1,),
        in_specs=[pl.BlockSpec((NC, 2, GA, F), lambda i: (0, 0, 0, 0))],
        out_specs=pl.BlockSpec((G, F), lambda i: (0, 0)),
        out_shape=jax.ShapeDtypeStruct((G, F), jnp.float32),
    )(pp)


def kernel(x, edge_index, batch, W1, b1, W2, b2):
    batp = jnp.concatenate([batch, jnp.full((N_PAD - N,), G, jnp.int32)])
    aux = jnp.concatenate([jnp.zeros((RPT, F), jnp.float32),
                           jnp.ones((CH, F), jnp.float32)], axis=0)
    eye8 = jnp.eye(8, dtype=jnp.float32)
    w1k = jnp.kron(eye8, W1)
    w2k = jnp.kron(eye8, W2)
    b1l = jnp.tile(b1, 8).reshape(1, 128)
    b2l = jnp.tile(b2, 8).reshape(1, 128)
    xlin = jnp.concatenate(
        [x, jnp.zeros((N_PAD - N, 5), jnp.float32)]).reshape(R_PAD, 40)

    ei3 = edge_index.reshape(2, NCHT, CH)

    hist = _sc_hist(ei3, aux)
    histl = hist.reshape(NC, R_PAD, 128)
    dis, z1 = _tc_pre(histl, xlin, w1k)
    p1 = _sc_agg(z1.reshape(N_PAD, F), ei3, aux)
    z2 = _tc_mid(p1.reshape(NC, R_PAD, 128), z1, dis, w2k, b1l)
    p2 = _sc_agg(z2.reshape(N_PAD, F), ei3, aux)
    h2 = _tc_fin(p2.reshape(NC, R_PAD, 128), z2, dis, b2l)
    pp = _sc_pool(h2.reshape(N_PAD, F), batp, aux)
    return _tc_div(pp)
